# baseline (speedup 1.0000x reference)
import functools

import jax
import jax.numpy as jnp
from jax import lax
from jax.experimental import pallas as pl
from jax.experimental.pallas import tpu as pltpu
from jax.experimental.pallas import tpu_sc as plsc

N = 10000
D = 128
E = 160000
EPS_LN = 1e-5

NT = 32
CH = 128
NCH = 40
E_PAD = NT * NCH * CH
STRIPE = 640
N_ACC = 16 * STRIPE
TRASH = 10008

RB = 1000
NB = N // RB
TB = 1000
NTB = N // TB

@functools.cache
def _mesh():
    return plsc.VectorSubcoreMesh(core_axis_name="c", subcore_axis_name="s")


def _sc_segment_sum(h, src3, dst3, zer128):
    @functools.partial(
        pl.kernel,
        out_type=jax.ShapeDtypeStruct((2, N_ACC, D), jnp.float32),
        mesh=_mesh(),
        scratch_types=[
            pltpu.VMEM((NCH, CH), jnp.int32),
            pltpu.VMEM((NCH, CH), jnp.int32),
            pltpu.VMEM((CH, D), jnp.float32),
            pltpu.VMEM_SHARED((N_ACC, D), jnp.float32),
            pltpu.SemaphoreType.DMA,
        ],
    )
    def k(h_h, src_h, dst_h, zer_h, out_h,
          idxs_v, idxd_v, rows_v, agg_sh, sem):
        cid = lax.axis_index("c")
        sid = lax.axis_index("s")
        tg = cid * 16 + sid
        pltpu.sync_copy(src_h.at[tg], idxs_v)
        pltpu.sync_copy(dst_h.at[tg], idxd_v)
        pltpu.sync_copy(zer_h, rows_v)
        for p in range(STRIPE // CH):
            pltpu.sync_copy(rows_v, agg_sh.at[pl.ds(sid * STRIPE + p * CH, CH)])
        plsc.subcore_barrier()

        def body(j, carry):
            pltpu.async_copy(h_h.at[idxs_v.at[j]], rows_v, sem).wait()
            pltpu.sync_copy(rows_v, agg_sh.at[idxd_v.at[j]], add=True)
            return carry

        lax.fori_loop(0, NCH, body, 0)
        plsc.subcore_barrier()
        for p in range(STRIPE // CH):
            sl = pl.ds(sid * STRIPE + p * CH, CH)
            pltpu.sync_copy(agg_sh.at[sl], rows_v)
            pltpu.sync_copy(rows_v, out_h.at[cid, sl])

    return k(h, src3, dst3, zer128)


def _sc_edge_dots(q, s3, d3):
    @functools.partial(
        pl.kernel,
        out_type=jax.ShapeDtypeStruct((NT, NCH, CH), jnp.float32),
        mesh=_mesh(),
        scratch_types=[
            pltpu.VMEM((NCH, CH), jnp.int32),
            pltpu.VMEM((NCH, CH), jnp.int32),
            pltpu.VMEM((CH, D), jnp.float32),
            pltpu.VMEM((CH, D), jnp.float32),
            pltpu.VMEM((NCH, CH), jnp.float32),
            pltpu.SemaphoreType.DMA,
        ],
        compiler_params=pltpu.CompilerParams(needs_layout_passes=False),
    )
    def k(q_h, s_h, d_h, out_h, idxs_v, idxd_v, rs_v, rd_v, dots_v, sem):
        cid = lax.axis_index("c")
        sid = lax.axis_index("s")
        tg = cid * 16 + sid
        pltpu.sync_copy(s_h.at[tg], idxs_v)
        pltpu.sync_copy(d_h.at[tg], idxd_v)
        lanes = lax.iota(jnp.int32, 16)

        def chunk(j, carry):
            cs = pltpu.async_copy(q_h.at[idxs_v.at[j]], rs_v, sem)
            cd = pltpu.async_copy(q_h.at[idxd_v.at[j]], rd_v, sem)
            cs.wait()
            cd.wait()

            def group(g, carry2):
                acc = jnp.zeros((16,), jnp.float32)
                for i in range(16):
                    r = g * 16 + i
                    p = jnp.zeros((16,), jnp.float32)
                    for kk in range(D // 16):
                        sl = pl.ds(16 * kk, 16)
                        p = p + rs_v[r, sl] * rd_v[r, sl]
                    acc = jnp.where(lanes == i, jnp.sum(p), acc)
                dots_v[j, pl.ds(16 * g, 16)] = acc
                return carry2

            lax.fori_loop(0, CH // 16, group, 0)
            return carry

        lax.fori_loop(0, NCH, chunk, 0)
        pltpu.sync_copy(dots_v, out_h.at[tg])

    return k(q, s3, d3)


def _deg_rs(dref):
    s = dref[0, :, 0:1] + dref[1, :, 0:1]
    return lax.rsqrt(jnp.maximum(s, 1.0))


def _tc_prescale(feats, degO_p):
    def body(f_ref, dO_ref, o_ref):
        o_ref[...] = f_ref[...] * _deg_rs(dO_ref)

    return pl.pallas_call(
        body,
        grid=(NB,),
        in_specs=[
            pl.BlockSpec((RB, D), lambda t: (t, 0)),
            pl.BlockSpec((2, RB, 128), lambda t: (0, t, 0)),
        ],
        out_specs=pl.BlockSpec((RB, D), lambda t: (t, 0)),
        out_shape=jax.ShapeDtypeStruct((N, D), jnp.float32),
    )(feats, degO_p)


def _tc_layer1(agg_p, degI_p, degO_p, W1, b1, gamma, beta):
    def body(a_ref, dI_ref, dO_ref, w_ref, b_ref, g_ref, be_ref,
             h1_ref, h1s_ref):
        x = (a_ref[0] + a_ref[1]) * _deg_rs(dI_ref)
        y = jnp.dot(x, w_ref[...], preferred_element_type=jnp.float32)
        y = jnp.maximum(y + b_ref[...], 0.0)
        mu = jnp.mean(y, axis=1, keepdims=True)
        var = jnp.mean((y - mu) ** 2, axis=1, keepdims=True)
        h1 = (y - mu) * lax.rsqrt(var + EPS_LN) * g_ref[...] + be_ref[...]
        h1_ref[...] = h1
        h1s_ref[...] = h1 * _deg_rs(dO_ref)

    return pl.pallas_call(
        body,
        grid=(NB,),
        in_specs=[
            pl.BlockSpec((2, RB, D), lambda t: (0, t, 0)),
            pl.BlockSpec((2, RB, 128), lambda t: (0, t, 0)),
            pl.BlockSpec((2, RB, 128), lambda t: (0, t, 0)),
            pl.BlockSpec((D, D), lambda t: (0, 0)),
            pl.BlockSpec((1, D), lambda t: (0, 0)),
            pl.BlockSpec((1, D), lambda t: (0, 0)),
            pl.BlockSpec((1, D), lambda t: (0, 0)),
        ],
        out_specs=[
            pl.BlockSpec((RB, D), lambda t: (t, 0)),
            pl.BlockSpec((RB, D), lambda t: (t, 0)),
        ],
        out_shape=[jax.ShapeDtypeStruct((N, D), jnp.float32),
                   jax.ShapeDtypeStruct((N, D), jnp.float32)],
    )(agg_p, degI_p, degO_p, W1, b1.reshape(1, D), gamma.reshape(1, D),
      beta.reshape(1, D))


def _tc_layer2(agg_p, degI_p, W2, b2, dW1, db1, dW2, db2):
    def body(a_ref, dI_ref, w_ref, b_ref, w1_ref, c1_ref, w2_ref, c2_ref,
             h2_ref, q_ref, fl_ref):
        t = pl.program_id(0)
        x = (a_ref[0] + a_ref[1]) * _deg_rs(dI_ref)
        h2 = jnp.dot(x, w_ref[...], preferred_element_type=jnp.float32)
        h2 = jnp.maximum(h2 + b_ref[...], 0.0)
        q = jnp.dot(h2, w1_ref[...], preferred_element_type=jnp.float32) + c1_ref[...]
        qn = jnp.dot(h2, w2_ref[...], preferred_element_type=jnp.float32) + c2_ref[...]
        h2_ref[...] = h2
        q_ref[...] = q

        @pl.when(t == 0)
        def _():
            fl_ref[...] = jnp.zeros_like(fl_ref)

        fl_ref[...] += jnp.sum((h2 - qn) ** 2).reshape(1, 1)

    return pl.pallas_call(
        body,
        grid=(NB,),
        in_specs=[
            pl.BlockSpec((2, RB, D), lambda t: (0, t, 0)),
            pl.BlockSpec((2, RB, 128), lambda t: (0, t, 0)),
            pl.BlockSpec((D, D), lambda t: (0, 0)),
            pl.BlockSpec((1, D), lambda t: (0, 0)),
            pl.BlockSpec((D, D), lambda t: (0, 0)),
            pl.BlockSpec((1, D), lambda t: (0, 0)),
            pl.BlockSpec((D, D), lambda t: (0, 0)),
            pl.BlockSpec((1, D), lambda t: (0, 0)),
        ],
        out_specs=[
            pl.BlockSpec((RB, D), lambda t: (t, 0)),
            pl.BlockSpec((RB, D), lambda t: (t, 0)),
            pl.BlockSpec((1, 1), lambda t: (0, 0)),
        ],
        out_shape=[jax.ShapeDtypeStruct((N, D), jnp.float32),
                   jax.ShapeDtypeStruct((N, D), jnp.float32),
                   jax.ShapeDtypeStruct((1, 1), jnp.float32)],
    )(agg_p, degI_p, W2, b2.reshape(1, D), dW1, db1.reshape(1, D), dW2,
      db2.reshape(1, D))


def _softplus(x):
    return jnp.maximum(x, 0.0) + jnp.log1p(jnp.exp(-jnp.abs(x)))


def _tc_tri_loss(q, pairs):
    def body(p_ref, qi_ref, qj_ref, acc_ref):
        t = pl.program_id(0)
        bi = p_ref[0, t]
        bj = p_ref[1, t]
        a = lax.dot_general(qi_ref[...], qj_ref[...],
                            (((1,), (1,)), ((), ())),
                            preferred_element_type=jnp.float32)
        sp = _softplus(a)
        r = lax.broadcasted_iota(jnp.int32, (TB, TB), 0)
        c = lax.broadcasted_iota(jnp.int32, (TB, TB), 1)
        keep = jnp.logical_or(bi != bj, r < c)
        sp = jnp.where(keep, sp, 0.0)

        @pl.when(t == 0)
        def _():
            acc_ref[...] = jnp.zeros_like(acc_ref)

        acc_ref[...] += jnp.sum(sp).reshape(1, 1)

    npairs = pairs.shape[1]
    grid_spec = pltpu.PrefetchScalarGridSpec(
        num_scalar_prefetch=1,
        grid=(npairs,),
        in_specs=[
            pl.BlockSpec((TB, D), lambda t, p: (p[0, t], 0)),
            pl.BlockSpec((TB, D), lambda t, p: (p[1, t], 0)),
        ],
        out_specs=pl.BlockSpec((1, 1), lambda t, p: (0, 0)),
    )
    return pl.pallas_call(
        body,
        grid_spec=grid_spec,
        out_shape=jax.ShapeDtypeStruct((1, 1), jnp.float32),
    )(pairs, q, q)


def _tc_edge_terms(dots2, mask2):
    nrow = dots2.shape[0]
    blk = 128
    steps = nrow // blk

    def body(a_ref, m_ref, sp_ref, sd_ref, c_ref):
        t = pl.program_id(0)
        a = a_ref[...]
        m = m_ref[...]

        @pl.when(t == 0)
        def _():
            sp_ref[...] = jnp.zeros_like(sp_ref)
            sd_ref[...] = jnp.zeros_like(sd_ref)
            c_ref[...] = jnp.zeros_like(c_ref)

        sp_ref[...] += jnp.sum(m * _softplus(a)).reshape(1, 1)
        sd_ref[...] += jnp.sum(m * a).reshape(1, 1)
        c_ref[...] += jnp.sum(m).reshape(1, 1)

    return pl.pallas_call(
        body,
        grid=(steps,),
        in_specs=[
            pl.BlockSpec((blk, 128), lambda t: (t, 0)),
            pl.BlockSpec((blk, 128), lambda t: (t, 0)),
        ],
        out_specs=[
            pl.BlockSpec((1, 1), lambda t: (0, 0)),
            pl.BlockSpec((1, 1), lambda t: (0, 0)),
            pl.BlockSpec((1, 1), lambda t: (0, 0)),
        ],
        out_shape=[jax.ShapeDtypeStruct((1, 1), jnp.float32)] * 3,
    )(dots2, mask2)


def kernel(feats, edge_index, W1, b1, W2, b2, gamma, beta, dW1, db1, dW2, db2):
    src = edge_index[0]
    dst = edge_index[1]
    pad = E_PAD - E

    trash = jnp.full((pad,), TRASH, jnp.int32)
    src_deg3 = jnp.concatenate([src, trash]).reshape(NT, NCH, CH)
    dst_deg3 = jnp.concatenate([dst, trash]).reshape(NT, NCH, CH)
    src_gat3 = jnp.concatenate([src, jnp.zeros((pad,), jnp.int32)]).reshape(NT, NCH, CH)

    big = jnp.int32(2147483647)
    key = jnp.where(src < dst, src * N + dst, big)
    ks = jnp.sort(key)
    valid_s = ks < big
    s2 = jnp.where(valid_s, ks // N, 0)
    d2 = jnp.where(valid_s, ks % N, 0)
    first = jnp.concatenate([jnp.ones((1,), bool), ks[1:] != ks[:-1]])
    uniq = (valid_s & first).astype(jnp.float32)
    zpad = jnp.zeros((pad,), jnp.int32)
    s23 = jnp.concatenate([s2, zpad]).reshape(NT, NCH, CH)
    d23 = jnp.concatenate([d2, zpad]).reshape(NT, NCH, CH)
    mask2 = jnp.concatenate([uniq, jnp.zeros((pad,), jnp.float32)]).reshape(E_PAD // 128, 128)

    zer128 = jnp.zeros((CH, D), jnp.float32)

    ones_t = jnp.ones((N, D), jnp.float32)
    degO_p = _sc_segment_sum(ones_t, src_gat3, src_deg3, zer128)
    degI_p = _sc_segment_sum(ones_t, src_gat3, dst_deg3, zer128)

    h0s = _tc_prescale(feats, degO_p)
    agg1 = _sc_segment_sum(h0s, src_gat3, dst_deg3, zer128)
    h1, h1s = _tc_layer1(agg1, degI_p, degO_p, W1, b1, gamma, beta)
    agg2 = _sc_segment_sum(h1s, src_gat3, dst_deg3, zer128)
    h2, q, fl = _tc_layer2(agg2, degI_p, W2, b2, dW1, db1, dW2, db2)

    pairs = jnp.array([[bi for bi in range(NTB) for bj in range(bi, NTB)],
                       [bj for bi in range(NTB) for bj in range(bi, NTB)]],
                      dtype=jnp.int32)
    s_all = _tc_tri_loss(q, pairs)[0, 0]

    dots = _sc_edge_dots(q, s23, d23)
    spos, sdot, cnt = _tc_edge_terms(dots.reshape(E_PAD // 128, 128), mask2)
    spos = spos[0, 0]
    sneg = spos - sdot[0, 0]
    num_edges = cnt[0, 0]

    num_possible = N * N / 2.0
    pos_weight = (num_possible - num_edges) / (num_edges + 1e-6)
    count = N * (N - 1) / 2.0
    edge_loss = (s_all + pos_weight * sneg - spos) / count
    feature_rec_loss = fl[0, 0] / (N * D)
    loss = feature_rec_loss + edge_loss * 100.0
    return (h1, h2, q, h2, loss)


# speedup vs baseline: 1.0051x; 1.0051x over previous
import functools

import jax
import jax.numpy as jnp
from jax import lax
from jax.experimental import pallas as pl
from jax.experimental.pallas import tpu as pltpu
from jax.experimental.pallas import tpu_sc as plsc

N = 10000
D = 128
E = 160000
EPS_LN = 1e-5

NT = 32
CH = 128
NCH = 40
E_PAD = NT * NCH * CH
STRIPE = 640
N_ACC = 16 * STRIPE
TRASH = 10008

RB = 1000
NB = N // RB
TB = 1000
NTB = N // TB

@functools.cache
def _mesh():
    return plsc.VectorSubcoreMesh(core_axis_name="c", subcore_axis_name="s")


def _sc_segment_sum(h, src3, dst3, zer128):
    @functools.partial(
        pl.kernel,
        out_type=jax.ShapeDtypeStruct((2, N_ACC, D), jnp.float32),
        mesh=_mesh(),
        scratch_types=[
            pltpu.VMEM((NCH, CH), jnp.int32),
            pltpu.VMEM((NCH, CH), jnp.int32),
            pltpu.VMEM((CH, D), jnp.float32),
            pltpu.VMEM_SHARED((N_ACC, D), jnp.float32),
            pltpu.SemaphoreType.DMA,
        ],
    )
    def k(h_h, src_h, dst_h, zer_h, out_h,
          idxs_v, idxd_v, rows_v, agg_sh, sem):
        cid = lax.axis_index("c")
        sid = lax.axis_index("s")
        tg = cid * 16 + sid
        pltpu.sync_copy(src_h.at[tg], idxs_v)
        pltpu.sync_copy(dst_h.at[tg], idxd_v)
        pltpu.sync_copy(zer_h, rows_v)
        for p in range(STRIPE // CH):
            pltpu.sync_copy(rows_v, agg_sh.at[pl.ds(sid * STRIPE + p * CH, CH)])
        plsc.subcore_barrier()

        def body(j, carry):
            pltpu.async_copy(h_h.at[idxs_v.at[j]], rows_v, sem).wait()
            pltpu.sync_copy(rows_v, agg_sh.at[idxd_v.at[j]], add=True)
            return carry

        lax.fori_loop(0, NCH, body, 0)
        plsc.subcore_barrier()
        for p in range(STRIPE // CH):
            sl = pl.ds(sid * STRIPE + p * CH, CH)
            pltpu.sync_copy(agg_sh.at[sl], rows_v)
            pltpu.sync_copy(rows_v, out_h.at[cid, sl])

    return k(h, src3, dst3, zer128)


def _sc_histogram(idx3, ones128, zer128):
    @functools.partial(
        pl.kernel,
        out_type=jax.ShapeDtypeStruct((2, N_ACC, D), jnp.float32),
        mesh=_mesh(),
        scratch_types=[
            pltpu.VMEM((NCH, CH), jnp.int32),
            pltpu.VMEM((CH, D), jnp.float32),
            pltpu.VMEM((CH, D), jnp.float32),
            pltpu.VMEM_SHARED((N_ACC, D), jnp.float32),
        ],
    )
    def k(idx_h, ones_h, zer_h, out_h, idx_v, ones_v, stg_v, agg_sh):
        cid = lax.axis_index("c")
        sid = lax.axis_index("s")
        tg = cid * 16 + sid
        pltpu.sync_copy(idx_h.at[tg], idx_v)
        pltpu.sync_copy(ones_h, ones_v)
        pltpu.sync_copy(zer_h, stg_v)
        for p in range(STRIPE // CH):
            pltpu.sync_copy(stg_v, agg_sh.at[pl.ds(sid * STRIPE + p * CH, CH)])
        plsc.subcore_barrier()

        def body(j, carry):
            pltpu.sync_copy(ones_v, agg_sh.at[idx_v.at[j]], add=True)
            return carry

        lax.fori_loop(0, NCH, body, 0)
        plsc.subcore_barrier()
        for p in range(STRIPE // CH):
            sl = pl.ds(sid * STRIPE + p * CH, CH)
            pltpu.sync_copy(agg_sh.at[sl], stg_v)
            pltpu.sync_copy(stg_v, out_h.at[cid, sl])

    return k(idx3, ones128, zer128)


def _sc_gather_pairs(q, s3, d3):
    @functools.partial(
        pl.kernel,
        out_type=(jax.ShapeDtypeStruct((E_PAD, D), jnp.float32),
                  jax.ShapeDtypeStruct((E_PAD, D), jnp.float32)),
        mesh=_mesh(),
        scratch_types=[
            pltpu.VMEM((NCH, CH), jnp.int32),
            pltpu.VMEM((NCH, CH), jnp.int32),
            pltpu.VMEM((CH, D), jnp.float32),
            pltpu.VMEM((CH, D), jnp.float32),
            pltpu.SemaphoreType.DMA,
        ],
    )
    def k(q_h, s_h, d_h, outs_h, outd_h, idxs_v, idxd_v, rs_v, rd_v, sem):
        cid = lax.axis_index("c")
        sid = lax.axis_index("s")
        tg = cid * 16 + sid
        pltpu.sync_copy(s_h.at[tg], idxs_v)
        pltpu.sync_copy(d_h.at[tg], idxd_v)

        def chunk(j, carry):
            base = tg * (NCH * CH) + j * CH
            cs = pltpu.async_copy(q_h.at[idxs_v.at[j]], rs_v, sem)
            cd = pltpu.async_copy(q_h.at[idxd_v.at[j]], rd_v, sem)
            cs.wait()
            cd.wait()
            pltpu.sync_copy(rs_v, outs_h.at[pl.ds(base, CH)])
            pltpu.sync_copy(rd_v, outd_h.at[pl.ds(base, CH)])
            return carry

        lax.fori_loop(0, NCH, chunk, 0)

    return k(q, s3, d3)


def _deg_rs(dref):
    s = dref[0, :, 0:1] + dref[1, :, 0:1]
    return lax.rsqrt(jnp.maximum(s, 1.0))


def _tc_prescale(feats, degO_p):
    def body(f_ref, dO_ref, o_ref):
        o_ref[...] = f_ref[...] * _deg_rs(dO_ref)

    return pl.pallas_call(
        body,
        grid=(NB,),
        in_specs=[
            pl.BlockSpec((RB, D), lambda t: (t, 0)),
            pl.BlockSpec((2, RB, 128), lambda t: (0, t, 0)),
        ],
        out_specs=pl.BlockSpec((RB, D), lambda t: (t, 0)),
        out_shape=jax.ShapeDtypeStruct((N, D), jnp.float32),
    )(feats, degO_p)


def _tc_layer1(agg_p, degI_p, degO_p, W1, b1, gamma, beta):
    def body(a_ref, dI_ref, dO_ref, w_ref, b_ref, g_ref, be_ref,
             h1_ref, h1s_ref):
        x = (a_ref[0] + a_ref[1]) * _deg_rs(dI_ref)
        y = jnp.dot(x, w_ref[...], preferred_element_type=jnp.float32)
        y = jnp.maximum(y + b_ref[...], 0.0)
        mu = jnp.mean(y, axis=1, keepdims=True)
        var = jnp.mean((y - mu) ** 2, axis=1, keepdims=True)
        h1 = (y - mu) * lax.rsqrt(var + EPS_LN) * g_ref[...] + be_ref[...]
        h1_ref[...] = h1
        h1s_ref[...] = h1 * _deg_rs(dO_ref)

    return pl.pallas_call(
        body,
        grid=(NB,),
        in_specs=[
            pl.BlockSpec((2, RB, D), lambda t: (0, t, 0)),
            pl.BlockSpec((2, RB, 128), lambda t: (0, t, 0)),
            pl.BlockSpec((2, RB, 128), lambda t: (0, t, 0)),
            pl.BlockSpec((D, D), lambda t: (0, 0)),
            pl.BlockSpec((1, D), lambda t: (0, 0)),
            pl.BlockSpec((1, D), lambda t: (0, 0)),
            pl.BlockSpec((1, D), lambda t: (0, 0)),
        ],
        out_specs=[
            pl.BlockSpec((RB, D), lambda t: (t, 0)),
            pl.BlockSpec((RB, D), lambda t: (t, 0)),
        ],
        out_shape=[jax.ShapeDtypeStruct((N, D), jnp.float32),
                   jax.ShapeDtypeStruct((N, D), jnp.float32)],
    )(agg_p, degI_p, degO_p, W1, b1.reshape(1, D), gamma.reshape(1, D),
      beta.reshape(1, D))


def _tc_layer2(agg_p, degI_p, W2, b2, dW1, db1, dW2, db2):
    def body(a_ref, dI_ref, w_ref, b_ref, w1_ref, c1_ref, w2_ref, c2_ref,
             h2_ref, q_ref, fl_ref):
        t = pl.program_id(0)
        x = (a_ref[0] + a_ref[1]) * _deg_rs(dI_ref)
        h2 = jnp.dot(x, w_ref[...], preferred_element_type=jnp.float32)
        h2 = jnp.maximum(h2 + b_ref[...], 0.0)
        q = jnp.dot(h2, w1_ref[...], preferred_element_type=jnp.float32) + c1_ref[...]
        qn = jnp.dot(h2, w2_ref[...], preferred_element_type=jnp.float32) + c2_ref[...]
        h2_ref[...] = h2
        q_ref[...] = q

        @pl.when(t == 0)
        def _():
            fl_ref[...] = jnp.zeros_like(fl_ref)

        fl_ref[...] += jnp.sum((h2 - qn) ** 2).reshape(1, 1)

    return pl.pallas_call(
        body,
        grid=(NB,),
        in_specs=[
            pl.BlockSpec((2, RB, D), lambda t: (0, t, 0)),
            pl.BlockSpec((2, RB, 128), lambda t: (0, t, 0)),
            pl.BlockSpec((D, D), lambda t: (0, 0)),
            pl.BlockSpec((1, D), lambda t: (0, 0)),
            pl.BlockSpec((D, D), lambda t: (0, 0)),
            pl.BlockSpec((1, D), lambda t: (0, 0)),
            pl.BlockSpec((D, D), lambda t: (0, 0)),
            pl.BlockSpec((1, D), lambda t: (0, 0)),
        ],
        out_specs=[
            pl.BlockSpec((RB, D), lambda t: (t, 0)),
            pl.BlockSpec((RB, D), lambda t: (t, 0)),
            pl.BlockSpec((1, 1), lambda t: (0, 0)),
        ],
        out_shape=[jax.ShapeDtypeStruct((N, D), jnp.float32),
                   jax.ShapeDtypeStruct((N, D), jnp.float32),
                   jax.ShapeDtypeStruct((1, 1), jnp.float32)],
    )(agg_p, degI_p, W2, b2.reshape(1, D), dW1, db1.reshape(1, D), dW2,
      db2.reshape(1, D))


def _softplus(x):
    return jnp.maximum(x, 0.0) + jnp.log1p(jnp.exp(-jnp.abs(x)))


def _tc_tri_loss(q, pairs):
    def body(p_ref, qi_ref, qj_ref, acc_ref):
        t = pl.program_id(0)
        bi = p_ref[0, t]
        bj = p_ref[1, t]
        a = lax.dot_general(qi_ref[...], qj_ref[...],
                            (((1,), (1,)), ((), ())),
                            preferred_element_type=jnp.float32)
        sp = _softplus(a)
        r = lax.broadcasted_iota(jnp.int32, (TB, TB), 0)
        c = lax.broadcasted_iota(jnp.int32, (TB, TB), 1)
        keep = jnp.logical_or(bi != bj, r < c)
        sp = jnp.where(keep, sp, 0.0)

        @pl.when(t == 0)
        def _():
            acc_ref[...] = jnp.zeros_like(acc_ref)

        acc_ref[...] += jnp.sum(sp).reshape(1, 1)

    npairs = pairs.shape[1]
    grid_spec = pltpu.PrefetchScalarGridSpec(
        num_scalar_prefetch=1,
        grid=(npairs,),
        in_specs=[
            pl.BlockSpec((TB, D), lambda t, p: (p[0, t], 0)),
            pl.BlockSpec((TB, D), lambda t, p: (p[1, t], 0)),
        ],
        out_specs=pl.BlockSpec((1, 1), lambda t, p: (0, 0)),
    )
    return pl.pallas_call(
        body,
        grid_spec=grid_spec,
        out_shape=jax.ShapeDtypeStruct((1, 1), jnp.float32),
    )(pairs, q, q)


def _tc_edge_terms(qs, qd, mask_col):
    blk = 1024
    steps = E_PAD // blk

    def body(qs_ref, qd_ref, m_ref, sp_ref, sd_ref, c_ref):
        t = pl.program_id(0)
        prod = qs_ref[...] * qd_ref[...]
        ones_m = jnp.full((D, D), 1.0, jnp.float32)
        a = jnp.dot(prod, ones_m, preferred_element_type=jnp.float32)
        m = m_ref[...]

        @pl.when(t == 0)
        def _():
            sp_ref[...] = jnp.zeros_like(sp_ref)
            sd_ref[...] = jnp.zeros_like(sd_ref)
            c_ref[...] = jnp.zeros_like(c_ref)

        sp_ref[...] += (jnp.sum(m * _softplus(a)) / D).reshape(1, 1)
        sd_ref[...] += (jnp.sum(m * a) / D).reshape(1, 1)
        c_ref[...] += jnp.sum(m).reshape(1, 1)

    return pl.pallas_call(
        body,
        grid=(steps,),
        in_specs=[
            pl.BlockSpec((blk, D), lambda t: (t, 0)),
            pl.BlockSpec((blk, D), lambda t: (t, 0)),
            pl.BlockSpec((blk, 1), lambda t: (t, 0)),
        ],
        out_specs=[
            pl.BlockSpec((1, 1), lambda t: (0, 0)),
            pl.BlockSpec((1, 1), lambda t: (0, 0)),
            pl.BlockSpec((1, 1), lambda t: (0, 0)),
        ],
        out_shape=[jax.ShapeDtypeStruct((1, 1), jnp.float32)] * 3,
    )(qs, qd, mask_col)


def kernel(feats, edge_index, W1, b1, W2, b2, gamma, beta, dW1, db1, dW2, db2):
    src = edge_index[0]
    dst = edge_index[1]
    pad = E_PAD - E

    trash = jnp.full((pad,), TRASH, jnp.int32)
    src_deg3 = jnp.concatenate([src, trash]).reshape(NT, NCH, CH)
    dst_deg3 = jnp.concatenate([dst, trash]).reshape(NT, NCH, CH)
    src_gat3 = jnp.concatenate([src, jnp.zeros((pad,), jnp.int32)]).reshape(NT, NCH, CH)

    big = jnp.int32(2147483647)
    key = jnp.where(src < dst, src * N + dst, big)
    ks = jnp.sort(key)
    valid_s = ks < big
    s2 = jnp.where(valid_s, ks // N, 0)
    d2 = jnp.where(valid_s, ks % N, 0)
    first = jnp.concatenate([jnp.ones((1,), bool), ks[1:] != ks[:-1]])
    uniq = (valid_s & first).astype(jnp.float32)
    zpad = jnp.zeros((pad,), jnp.int32)
    s23 = jnp.concatenate([s2, zpad]).reshape(NT, NCH, CH)
    d23 = jnp.concatenate([d2, zpad]).reshape(NT, NCH, CH)
    mask_col = jnp.concatenate([uniq, jnp.zeros((pad,), jnp.float32)]).reshape(E_PAD, 1)

    zer128 = jnp.zeros((CH, D), jnp.float32)
    ones128 = jnp.ones((CH, D), jnp.float32)

    degO_p = _sc_histogram(src_deg3, ones128, zer128)
    degI_p = _sc_histogram(dst_deg3, ones128, zer128)

    h0s = _tc_prescale(feats, degO_p)
    agg1 = _sc_segment_sum(h0s, src_gat3, dst_deg3, zer128)
    h1, h1s = _tc_layer1(agg1, degI_p, degO_p, W1, b1, gamma, beta)
    agg2 = _sc_segment_sum(h1s, src_gat3, dst_deg3, zer128)
    h2, q, fl = _tc_layer2(agg2, degI_p, W2, b2, dW1, db1, dW2, db2)

    pairs = jnp.array([[bi for bi in range(NTB) for bj in range(bi, NTB)],
                       [bj for bi in range(NTB) for bj in range(bi, NTB)]],
                      dtype=jnp.int32)
    s_all = _tc_tri_loss(q, pairs)[0, 0]

    qs_rows, qd_rows = _sc_gather_pairs(q, s23, d23)
    spos, sdot, cnt = _tc_edge_terms(qs_rows, qd_rows, mask_col)
    spos = spos[0, 0]
    sneg = spos - sdot[0, 0]
    num_edges = cnt[0, 0]

    num_possible = N * N / 2.0
    pos_weight = (num_possible - num_edges) / (num_edges + 1e-6)
    count = N * (N - 1) / 2.0
    edge_loss = (s_all + pos_weight * sneg - spos) / count
    feature_rec_loss = fl[0, 0] / (N * D)
    loss = feature_rec_loss + edge_loss * 100.0
    return (h1, h2, q, h2, loss)


# speedup vs baseline: 8.3244x; 8.2818x over previous
import functools

import jax
import jax.numpy as jnp
from jax import lax
from jax.experimental import pallas as pl
from jax.experimental.pallas import tpu as pltpu
from jax.experimental.pallas import tpu_sc as plsc

N = 10000
D = 128
E = 160000
EPS_LN = 1e-5

NT = 32
CH = 128
NCH = 40
E_PAD = NT * NCH * CH
STRIPE = 640
N_ACC = 16 * STRIPE
TRASH = 10008

RB = 1000
NB = N // RB
TB = 1000
NTB = N // TB

@functools.cache
def _mesh():
    return plsc.VectorSubcoreMesh(core_axis_name="c", subcore_axis_name="s")


def _sc_segment_sum(h, src3, dst3, zer128):
    @functools.partial(
        pl.kernel,
        out_type=jax.ShapeDtypeStruct((2, N_ACC, D), jnp.float32),
        mesh=_mesh(),
        scratch_types=[
            pltpu.VMEM((NCH, CH), jnp.int32),
            pltpu.VMEM((NCH, CH), jnp.int32),
            pltpu.VMEM((CH, D), jnp.float32),
            pltpu.VMEM_SHARED((N_ACC, D), jnp.float32),
            pltpu.SemaphoreType.DMA,
        ],
    )
    def k(h_h, src_h, dst_h, zer_h, out_h,
          idxs_v, idxd_v, rows_v, agg_sh, sem):
        cid = lax.axis_index("c")
        sid = lax.axis_index("s")
        tg = cid * 16 + sid
        pltpu.sync_copy(src_h.at[tg], idxs_v)
        pltpu.sync_copy(dst_h.at[tg], idxd_v)
        pltpu.sync_copy(zer_h, rows_v)
        for p in range(STRIPE // CH):
            pltpu.sync_copy(rows_v, agg_sh.at[pl.ds(sid * STRIPE + p * CH, CH)])
        plsc.subcore_barrier()

        def body(j, carry):
            pltpu.async_copy(h_h.at[idxs_v.at[j]], rows_v, sem).wait()
            pltpu.sync_copy(rows_v, agg_sh.at[idxd_v.at[j]], add=True)
            return carry

        lax.fori_loop(0, NCH, body, 0)
        plsc.subcore_barrier()
        for p in range(STRIPE // CH):
            sl = pl.ds(sid * STRIPE + p * CH, CH)
            pltpu.sync_copy(agg_sh.at[sl], rows_v)
            pltpu.sync_copy(rows_v, out_h.at[cid, sl])

    return k(h, src3, dst3, zer128)


def _sc_histogram(idx3, ones128, zer128):
    @functools.partial(
        pl.kernel,
        out_type=jax.ShapeDtypeStruct((2, N_ACC, D), jnp.float32),
        mesh=_mesh(),
        scratch_types=[
            pltpu.VMEM((NCH, CH), jnp.int32),
            pltpu.VMEM((CH, D), jnp.float32),
            pltpu.VMEM((CH, D), jnp.float32),
            pltpu.VMEM_SHARED((N_ACC, D), jnp.float32),
        ],
    )
    def k(idx_h, ones_h, zer_h, out_h, idx_v, ones_v, stg_v, agg_sh):
        cid = lax.axis_index("c")
        sid = lax.axis_index("s")
        tg = cid * 16 + sid
        pltpu.sync_copy(idx_h.at[tg], idx_v)
        pltpu.sync_copy(ones_h, ones_v)
        pltpu.sync_copy(zer_h, stg_v)
        for p in range(STRIPE // CH):
            pltpu.sync_copy(stg_v, agg_sh.at[pl.ds(sid * STRIPE + p * CH, CH)])
        plsc.subcore_barrier()

        def body(j, carry):
            pltpu.sync_copy(ones_v, agg_sh.at[idx_v.at[j]], add=True)
            return carry

        lax.fori_loop(0, NCH, body, 0)
        plsc.subcore_barrier()
        for p in range(STRIPE // CH):
            sl = pl.ds(sid * STRIPE + p * CH, CH)
            pltpu.sync_copy(agg_sh.at[sl], stg_v)
            pltpu.sync_copy(stg_v, out_h.at[cid, sl])

    return k(idx3, ones128, zer128)


def _sc_gather_pairs(q, s3, d3):
    @functools.partial(
        pl.kernel,
        out_type=(jax.ShapeDtypeStruct((E_PAD, D), jnp.float32),
                  jax.ShapeDtypeStruct((E_PAD, D), jnp.float32)),
        mesh=_mesh(),
        scratch_types=[
            pltpu.VMEM((NCH, CH), jnp.int32),
            pltpu.VMEM((NCH, CH), jnp.int32),
            pltpu.VMEM((CH, D), jnp.float32),
            pltpu.VMEM((CH, D), jnp.float32),
            pltpu.SemaphoreType.DMA,
        ],
    )
    def k(q_h, s_h, d_h, outs_h, outd_h, idxs_v, idxd_v, rs_v, rd_v, sem):
        cid = lax.axis_index("c")
        sid = lax.axis_index("s")
        tg = cid * 16 + sid
        pltpu.sync_copy(s_h.at[tg], idxs_v)
        pltpu.sync_copy(d_h.at[tg], idxd_v)

        def chunk(j, carry):
            base = tg * (NCH * CH) + j * CH
            cs = pltpu.async_copy(q_h.at[idxs_v.at[j]], rs_v, sem)
            cd = pltpu.async_copy(q_h.at[idxd_v.at[j]], rd_v, sem)
            cs.wait()
            cd.wait()
            pltpu.sync_copy(rs_v, outs_h.at[pl.ds(base, CH)])
            pltpu.sync_copy(rd_v, outd_h.at[pl.ds(base, CH)])
            return carry

        lax.fori_loop(0, NCH, chunk, 0)

    return k(q, s3, d3)


def _deg_rs(dref):
    s = dref[0, :, 0:1] + dref[1, :, 0:1]
    return lax.rsqrt(jnp.maximum(s, 1.0))


def _tc_prescale(feats, degO_p):
    def body(f_ref, dO_ref, o_ref):
        o_ref[...] = f_ref[...] * _deg_rs(dO_ref)

    return pl.pallas_call(
        body,
        grid=(NB,),
        in_specs=[
            pl.BlockSpec((RB, D), lambda t: (t, 0)),
            pl.BlockSpec((2, RB, 128), lambda t: (0, t, 0)),
        ],
        out_specs=pl.BlockSpec((RB, D), lambda t: (t, 0)),
        out_shape=jax.ShapeDtypeStruct((N, D), jnp.float32),
    )(feats, degO_p)


def _tc_layer1(agg_p, degI_p, degO_p, W1, b1, gamma, beta):
    def body(a_ref, dI_ref, dO_ref, w_ref, b_ref, g_ref, be_ref,
             h1_ref, h1s_ref):
        x = (a_ref[0] + a_ref[1]) * _deg_rs(dI_ref)
        y = jnp.dot(x, w_ref[...], preferred_element_type=jnp.float32)
        y = jnp.maximum(y + b_ref[...], 0.0)
        mu = jnp.mean(y, axis=1, keepdims=True)
        var = jnp.mean((y - mu) ** 2, axis=1, keepdims=True)
        h1 = (y - mu) * lax.rsqrt(var + EPS_LN) * g_ref[...] + be_ref[...]
        h1_ref[...] = h1
        h1s_ref[...] = h1 * _deg_rs(dO_ref)

    return pl.pallas_call(
        body,
        grid=(NB,),
        in_specs=[
            pl.BlockSpec((2, RB, D), lambda t: (0, t, 0)),
            pl.BlockSpec((2, RB, 128), lambda t: (0, t, 0)),
            pl.BlockSpec((2, RB, 128), lambda t: (0, t, 0)),
            pl.BlockSpec((D, D), lambda t: (0, 0)),
            pl.BlockSpec((1, D), lambda t: (0, 0)),
            pl.BlockSpec((1, D), lambda t: (0, 0)),
            pl.BlockSpec((1, D), lambda t: (0, 0)),
        ],
        out_specs=[
            pl.BlockSpec((RB, D), lambda t: (t, 0)),
            pl.BlockSpec((RB, D), lambda t: (t, 0)),
        ],
        out_shape=[jax.ShapeDtypeStruct((N, D), jnp.float32),
                   jax.ShapeDtypeStruct((N, D), jnp.float32)],
    )(agg_p, degI_p, degO_p, W1, b1.reshape(1, D), gamma.reshape(1, D),
      beta.reshape(1, D))


def _tc_layer2(agg_p, degI_p, W2, b2, dW1, db1, dW2, db2):
    def body(a_ref, dI_ref, w_ref, b_ref, w1_ref, c1_ref, w2_ref, c2_ref,
             h2_ref, q_ref, fl_ref):
        t = pl.program_id(0)
        x = (a_ref[0] + a_ref[1]) * _deg_rs(dI_ref)
        h2 = jnp.dot(x, w_ref[...], preferred_element_type=jnp.float32)
        h2 = jnp.maximum(h2 + b_ref[...], 0.0)
        q = jnp.dot(h2, w1_ref[...], preferred_element_type=jnp.float32) + c1_ref[...]
        qn = jnp.dot(h2, w2_ref[...], preferred_element_type=jnp.float32) + c2_ref[...]
        h2_ref[...] = h2
        q_ref[...] = q

        @pl.when(t == 0)
        def _():
            fl_ref[...] = jnp.zeros_like(fl_ref)

        fl_ref[...] += jnp.sum((h2 - qn) ** 2).reshape(1, 1)

    return pl.pallas_call(
        body,
        grid=(NB,),
        in_specs=[
            pl.BlockSpec((2, RB, D), lambda t: (0, t, 0)),
            pl.BlockSpec((2, RB, 128), lambda t: (0, t, 0)),
            pl.BlockSpec((D, D), lambda t: (0, 0)),
            pl.BlockSpec((1, D), lambda t: (0, 0)),
            pl.BlockSpec((D, D), lambda t: (0, 0)),
            pl.BlockSpec((1, D), lambda t: (0, 0)),
            pl.BlockSpec((D, D), lambda t: (0, 0)),
            pl.BlockSpec((1, D), lambda t: (0, 0)),
        ],
        out_specs=[
            pl.BlockSpec((RB, D), lambda t: (t, 0)),
            pl.BlockSpec((RB, D), lambda t: (t, 0)),
            pl.BlockSpec((1, 1), lambda t: (0, 0)),
        ],
        out_shape=[jax.ShapeDtypeStruct((N, D), jnp.float32),
                   jax.ShapeDtypeStruct((N, D), jnp.float32),
                   jax.ShapeDtypeStruct((1, 1), jnp.float32)],
    )(agg_p, degI_p, W2, b2.reshape(1, D), dW1, db1.reshape(1, D), dW2,
      db2.reshape(1, D))


def _softplus(x):
    return jnp.maximum(x, 0.0) + jnp.log1p(jnp.exp(-jnp.abs(x)))


def _tc_tri_loss(q, pairs):
    def body(p_ref, qi_ref, qj_ref, acc_ref):
        t = pl.program_id(0)
        bi = p_ref[0, t]
        bj = p_ref[1, t]
        a = lax.dot_general(qi_ref[...], qj_ref[...],
                            (((1,), (1,)), ((), ())),
                            preferred_element_type=jnp.float32)
        sp = _softplus(a)
        r = lax.broadcasted_iota(jnp.int32, (TB, TB), 0)
        c = lax.broadcasted_iota(jnp.int32, (TB, TB), 1)
        keep = jnp.logical_or(bi != bj, r < c)
        sp = jnp.where(keep, sp, 0.0)

        @pl.when(t == 0)
        def _():
            acc_ref[...] = jnp.zeros_like(acc_ref)

        acc_ref[...] += jnp.sum(sp).reshape(1, 1)

    npairs = pairs.shape[1]
    grid_spec = pltpu.PrefetchScalarGridSpec(
        num_scalar_prefetch=1,
        grid=(npairs,),
        in_specs=[
            pl.BlockSpec((TB, D), lambda t, p: (p[0, t], 0)),
            pl.BlockSpec((TB, D), lambda t, p: (p[1, t], 0)),
        ],
        out_specs=pl.BlockSpec((1, 1), lambda t, p: (0, 0)),
    )
    return pl.pallas_call(
        body,
        grid_spec=grid_spec,
        out_shape=jax.ShapeDtypeStruct((1, 1), jnp.float32),
    )(pairs, q, q)


def _tc_edge_terms(qs, qd, mask_col):
    blk = 1024
    steps = E_PAD // blk

    def body(qs_ref, qd_ref, m_ref, sp_ref, sd_ref, c_ref):
        t = pl.program_id(0)
        prod = qs_ref[...] * qd_ref[...]
        ones_m = jnp.full((D, D), 1.0, jnp.float32)
        a = jnp.dot(prod, ones_m, preferred_element_type=jnp.float32)
        m = m_ref[...]

        @pl.when(t == 0)
        def _():
            sp_ref[...] = jnp.zeros_like(sp_ref)
            sd_ref[...] = jnp.zeros_like(sd_ref)
            c_ref[...] = jnp.zeros_like(c_ref)

        sp_ref[...] += (jnp.sum(m * _softplus(a)) / D).reshape(1, 1)
        sd_ref[...] += (jnp.sum(m * a) / D).reshape(1, 1)
        c_ref[...] += jnp.sum(m).reshape(1, 1)

    return pl.pallas_call(
        body,
        grid=(steps,),
        in_specs=[
            pl.BlockSpec((blk, D), lambda t: (t, 0)),
            pl.BlockSpec((blk, D), lambda t: (t, 0)),
            pl.BlockSpec((blk, 1), lambda t: (t, 0)),
        ],
        out_specs=[
            pl.BlockSpec((1, 1), lambda t: (0, 0)),
            pl.BlockSpec((1, 1), lambda t: (0, 0)),
            pl.BlockSpec((1, 1), lambda t: (0, 0)),
        ],
        out_shape=[jax.ShapeDtypeStruct((1, 1), jnp.float32)] * 3,
    )(qs, qd, mask_col)


def kernel(feats, edge_index, W1, b1, W2, b2, gamma, beta, dW1, db1, dW2, db2):
    src = edge_index[0]
    dst = edge_index[1]
    pad = E_PAD - E

    trash = jnp.full((pad,), TRASH, jnp.int32)
    src_deg3 = jnp.concatenate([src, trash]).reshape(NT, NCH, CH)
    dst_deg3 = jnp.concatenate([dst, trash]).reshape(NT, NCH, CH)
    spread_pad = (jnp.arange(pad, dtype=jnp.int32) * 64) % N
    src_gat3 = jnp.concatenate([src, spread_pad]).reshape(NT, NCH, CH)

    big = jnp.int32(2147483647)
    key = jnp.where(src < dst, src * N + dst, big)
    ks = jnp.sort(key)
    valid_s = ks < big
    spread_e = (jnp.arange(E_PAD, dtype=jnp.int32) * 64) % N
    s2 = jnp.where(valid_s, ks // N, spread_e[:E])
    d2 = jnp.where(valid_s, ks % N, spread_e[:E])
    first = jnp.concatenate([jnp.ones((1,), bool), ks[1:] != ks[:-1]])
    uniq = (valid_s & first).astype(jnp.float32)
    s23 = jnp.concatenate([s2, spread_e[E:]]).reshape(NT, NCH, CH)
    d23 = jnp.concatenate([d2, spread_e[E:]]).reshape(NT, NCH, CH)
    mask_col = jnp.concatenate([uniq, jnp.zeros((pad,), jnp.float32)]).reshape(E_PAD, 1)

    zer128 = jnp.zeros((CH, D), jnp.float32)
    ones128 = jnp.ones((CH, D), jnp.float32)

    degO_p = _sc_histogram(src_deg3, ones128, zer128)
    degI_p = _sc_histogram(dst_deg3, ones128, zer128)

    h0s = _tc_prescale(feats, degO_p)
    agg1 = _sc_segment_sum(h0s, src_gat3, dst_deg3, zer128)
    h1, h1s = _tc_layer1(agg1, degI_p, degO_p, W1, b1, gamma, beta)
    agg2 = _sc_segment_sum(h1s, src_gat3, dst_deg3, zer128)
    h2, q, fl = _tc_layer2(agg2, degI_p, W2, b2, dW1, db1, dW2, db2)

    pairs = jnp.array([[bi for bi in range(NTB) for bj in range(bi, NTB)],
                       [bj for bi in range(NTB) for bj in range(bi, NTB)]],
                      dtype=jnp.int32)
    s_all = _tc_tri_loss(q, pairs)[0, 0]

    qs_rows, qd_rows = _sc_gather_pairs(q, s23, d23)
    spos, sdot, cnt = _tc_edge_terms(qs_rows, qd_rows, mask_col)
    spos = spos[0, 0]
    sneg = spos - sdot[0, 0]
    num_edges = cnt[0, 0]

    num_possible = N * N / 2.0
    pos_weight = (num_possible - num_edges) / (num_edges + 1e-6)
    count = N * (N - 1) / 2.0
    edge_loss = (s_all + pos_weight * sneg - spos) / count
    feature_rec_loss = fl[0, 0] / (N * D)
    loss = feature_rec_loss + edge_loss * 100.0
    return (h1, h2, q, h2, loss)
